# Initial kernel scaffold; baseline (speedup 1.0000x reference)
import jax
import jax.numpy as jnp
from jax.experimental import pallas as pl


def kernel(positions, hash_table):
    raise NotImplementedError("write your pallas kernel here")


# speedup vs baseline: 7.9619x; 7.9619x over previous
import functools

import numpy as np
import jax
import jax.numpy as jnp
from jax import lax
from jax.experimental import pallas as pl
from jax.experimental.pallas import tpu as pltpu
from jax.experimental.pallas import tpu_sc as plsc

N = 262144
NUM_LEVELS = 16
FEAT = 2
TABLE_SIZE = 2 ** 19
MASK = TABLE_SIZE - 1
GRAN = 16
NGRAN = TABLE_SIZE * FEAT // GRAN
_growth = np.exp((np.log(2048.0) - np.log(16.0)) / (NUM_LEVELS - 1))
RES = [min(int(16 * _growth ** l), 2048) for l in range(NUM_LEVELS)]

P1 = -1640531535
P2 = 805459861

NW = 32
PW = N // NW
CHUNK = 256
NCH = PW // CHUNK
G = CHUNK // 16


def _make_kernel():
    mesh = plsc.VectorSubcoreMesh(core_axis_name="c", subcore_axis_name="s")

    @functools.partial(
        pl.kernel,
        mesh=mesh,
        out_type=jax.ShapeDtypeStruct((N, 2 * NUM_LEVELS), jnp.float32),
        compiler_params=pltpu.CompilerParams(
            needs_layout_passes=False, use_tc_tiling_on_sc=False),
        scratch_types=[
            pltpu.VMEM((CHUNK,), jnp.float32),
            pltpu.VMEM((CHUNK,), jnp.float32),
            pltpu.VMEM((CHUNK,), jnp.float32),
            pltpu.VMEM((8 * CHUNK,), jnp.int32),
            pltpu.VMEM((8 * CHUNK,), jnp.int32),
            pltpu.VMEM((8 * CHUNK,), jnp.int32),
            pltpu.VMEM((8 * CHUNK,), jnp.int32),
            pltpu.VMEM((8 * CHUNK, GRAN), jnp.float32),
            pltpu.VMEM((8 * CHUNK, GRAN), jnp.float32),
            pltpu.VMEM((CHUNK, 2 * NUM_LEVELS), jnp.float32),
            pltpu.SemaphoreType.DMA,
            pltpu.SemaphoreType.DMA,
        ],
    )
    def hash_enc(xs_h, ys_h, zs_h, tab_h, out_h,
                 px, py, pz, idx0, idx1, col0b, col1b, rows0, rows1, outb,
                 sem0, sem1):
        wid = lax.axis_index("s") * 2 + lax.axis_index("c")
        ids = lax.iota(jnp.int32, 16)
        idxrefs = (idx0, idx1)
        colrefs = (col0b, col1b)
        rowrefs = (rows0, rows1)
        sems = (sem0, sem1)

        def compute_idx(level, idxref, colref):
            res = RES[level]
            s = np.float32(res - 1)
            rmax = jnp.int32(res - 1)
            zero_i = jnp.int32(0)
            one_i = jnp.int32(1)
            three_i = jnp.int32(3)
            seven_i = jnp.int32(7)

            def gbody(g, carry):
                o = g * jnp.int32(16)

                def corners1d(ref):
                    sc = ref[pl.ds(o, 16)] * s
                    i0 = sc.astype(jnp.int32)
                    c0 = jnp.minimum(jnp.maximum(i0, zero_i), rmax)
                    c1 = jnp.minimum(jnp.maximum(i0 + one_i, zero_i), rmax)
                    return c0, c1

                x0, x1 = corners1d(px)
                y0, y1 = corners1d(py)
                z0, z1 = corners1d(pz)
                p1 = jnp.int32(P1)
                p2 = jnp.int32(P2)
                hy0 = y0 * p1
                hy1 = y1 * p1
                hz0 = z0 * p2
                hz1 = z1 * p2
                corners = ((x0, hy0, hz0), (x0, hy0, hz1),
                           (x0, hy1, hz0), (x0, hy1, hz1),
                           (x1, hy0, hz0), (x1, hy0, hz1),
                           (x1, hy1, hz0), (x1, hy1, hz1))
                mask_i = jnp.int32(MASK)
                for c, (a, b, d) in enumerate(corners):
                    h = (a ^ b ^ d) & mask_i
                    off = o + jnp.int32(c * CHUNK)
                    idxref[pl.ds(off, 16)] = lax.shift_right_logical(h, three_i)
                    colref[pl.ds(off, 16)] = h & seven_i
                return carry

            lax.fori_loop(jnp.int32(0), jnp.int32(G), gbody, jnp.int32(0))

        def interp(level, colref, rowref):
            res = RES[level]
            s = np.float32(res - 1)

            def gbody(g, carry):
                o = g * jnp.int32(16)

                def weight1d(ref):
                    sc = ref[pl.ds(o, 16)] * s
                    return sc - sc.astype(jnp.int32).astype(jnp.float32)

                wx = weight1d(px)
                wy = weight1d(py)
                wz = weight1d(pz)
                one_f = np.float32(1.0)
                omx = one_f - wx
                omy = one_f - wy
                omz = one_f - wz
                one_i = jnp.int32(1)
                cf = []
                for c in range(8):
                    off = o + jnp.int32(c * CHUNK)
                    ridx = ids + off
                    colv = colref[pl.ds(off, 16)]
                    ca = colv + colv
                    cb = ca + one_i
                    cf.append((plsc.load_gather(rowref, [ridx, ca]),
                               plsc.load_gather(rowref, [ridx, cb])))
                pidx = ids + o
                for f in range(FEAT):
                    c00 = cf[0][f] * omx + cf[4][f] * wx
                    c01 = cf[1][f] * omx + cf[5][f] * wx
                    c10 = cf[2][f] * omx + cf[6][f] * wx
                    c11 = cf[3][f] * omx + cf[7][f] * wx
                    c0 = c00 * omy + c10 * wy
                    c1 = c01 * omy + c11 * wy
                    outv = c0 * omz + c1 * wz
                    col = jnp.full((16,), 2 * level + f, jnp.int32)
                    plsc.store_scatter(outb, [pidx, col], outv)
                return carry

            lax.fori_loop(jnp.int32(0), jnp.int32(G), gbody, jnp.int32(0))

        def chunk_body(ci, carry):
            base = wid * jnp.int32(PW) + ci * jnp.int32(CHUNK)
            pltpu.sync_copy(xs_h.at[pl.ds(base, CHUNK)], px)
            pltpu.sync_copy(ys_h.at[pl.ds(base, CHUNK)], py)
            pltpu.sync_copy(zs_h.at[pl.ds(base, CHUNK)], pz)

            def prep(g, c2):
                o = g * jnp.int32(16)
                for ref in (px, py, pz):
                    ref[pl.ds(o, 16)] = (
                        (ref[pl.ds(o, 16)] + np.float32(1.0)) * np.float32(0.5))
                return c2

            lax.fori_loop(jnp.int32(0), jnp.int32(G), prep, jnp.int32(0))

            compute_idx(0, idx0, col0b)
            cps = [None] * NUM_LEVELS
            cps[0] = pltpu.make_async_copy(tab_h.at[idx0], rows0, sem0)
            cps[0].start()
            for l in range(NUM_LEVELS):
                if l + 1 < NUM_LEVELS:
                    nb = (l + 1) % 2
                    compute_idx(l + 1, idxrefs[nb], colrefs[nb])
                    cps[l + 1] = pltpu.make_async_copy(
                        tab_h.at[idxrefs[nb]], rowrefs[nb], sems[nb])
                    cps[l + 1].start()
                b = l % 2
                cps[l].wait()
                interp(l, colrefs[b], rowrefs[b])
            pltpu.sync_copy(outb, out_h.at[pl.ds(base, CHUNK)])
            return carry

        lax.fori_loop(jnp.int32(0), jnp.int32(NCH), chunk_body, jnp.int32(0))

    return hash_enc


_KERNEL = _make_kernel()


def kernel(positions, hash_table):
    xs = positions[:, 0]
    ys = positions[:, 1]
    zs = positions[:, 2]
    tab = hash_table.reshape(NGRAN, GRAN)
    return _KERNEL(xs, ys, zs, tab)


# speedup vs baseline: 7.9687x; 1.0009x over previous
import functools

import numpy as np
import jax
import jax.numpy as jnp
from jax import lax
from jax.experimental import pallas as pl
from jax.experimental.pallas import tpu as pltpu
from jax.experimental.pallas import tpu_sc as plsc

N = 262144
NUM_LEVELS = 16
FEAT = 2
TABLE_SIZE = 2 ** 19
MASK = TABLE_SIZE - 1
GRAN = 16
NGRAN = TABLE_SIZE * FEAT // GRAN
_growth = np.exp((np.log(2048.0) - np.log(16.0)) / (NUM_LEVELS - 1))
RES = [min(int(16 * _growth ** l), 2048) for l in range(NUM_LEVELS)]

P1 = -1640531535
P2 = 805459861

NW = 32
PW = N // NW
CHUNK = 256
NCH = PW // CHUNK
G = CHUNK // 16


def _make_kernel():
    mesh = plsc.VectorSubcoreMesh(core_axis_name="c", subcore_axis_name="s")

    @functools.partial(
        pl.kernel,
        mesh=mesh,
        out_type=jax.ShapeDtypeStruct((N, 2 * NUM_LEVELS), jnp.float32),
        compiler_params=pltpu.CompilerParams(
            needs_layout_passes=False, use_tc_tiling_on_sc=False),
        scratch_types=[
            pltpu.VMEM((CHUNK,), jnp.float32),
            pltpu.VMEM((CHUNK,), jnp.float32),
            pltpu.VMEM((CHUNK,), jnp.float32),
            pltpu.VMEM((8 * CHUNK,), jnp.int32),
            pltpu.VMEM((8 * CHUNK,), jnp.int32),
            pltpu.VMEM((8 * CHUNK,), jnp.int32),
            pltpu.VMEM((8 * CHUNK,), jnp.int32),
            pltpu.VMEM((8 * CHUNK, GRAN), jnp.float32),
            pltpu.VMEM((8 * CHUNK, GRAN), jnp.float32),
            pltpu.VMEM((CHUNK, 2 * NUM_LEVELS), jnp.float32),
            pltpu.SemaphoreType.DMA,
            pltpu.SemaphoreType.DMA,
        ],
    )
    def hash_enc(xs_h, ys_h, zs_h, tab_h, out_h,
                 px, py, pz, idx0, idx1, col0b, col1b, rows0, rows1,
                 outb, sem0, sem1):
        wid = lax.axis_index("s") * 2 + lax.axis_index("c")
        ids = lax.iota(jnp.int32, 16)
        idxrefs = (idx0, idx1)
        colrefs = (col0b, col1b)
        rowrefs = (rows0, rows1)
        sems = (sem0, sem1)

        def compute_idx(level, idxref, colref):
            res = RES[level]
            s = np.float32(res - 1)
            rmax = jnp.int32(res - 1)
            zero_i = jnp.int32(0)
            one_i = jnp.int32(1)
            three_i = jnp.int32(3)
            seven_i = jnp.int32(7)

            def gbody(g, carry):
                o = g * jnp.int32(16)

                def corners1d(ref):
                    sc = ref[pl.ds(o, 16)] * s
                    i0 = sc.astype(jnp.int32)
                    c0 = jnp.minimum(jnp.maximum(i0, zero_i), rmax)
                    c1 = jnp.minimum(jnp.maximum(i0 + one_i, zero_i), rmax)
                    return c0, c1

                x0, x1 = corners1d(px)
                y0, y1 = corners1d(py)
                z0, z1 = corners1d(pz)
                p1 = jnp.int32(P1)
                p2 = jnp.int32(P2)
                hy0 = y0 * p1
                hy1 = y1 * p1
                hz0 = z0 * p2
                hz1 = z1 * p2
                corners = ((x0, hy0, hz0), (x0, hy0, hz1),
                           (x0, hy1, hz0), (x0, hy1, hz1),
                           (x1, hy0, hz0), (x1, hy0, hz1),
                           (x1, hy1, hz0), (x1, hy1, hz1))
                mask_i = jnp.int32(MASK)
                for c, (a, b, d) in enumerate(corners):
                    h = (a ^ b ^ d) & mask_i
                    off = o + jnp.int32(c * CHUNK)
                    idxref[pl.ds(off, 16)] = lax.shift_right_logical(h, three_i)
                    colref[pl.ds(off, 16)] = h & seven_i
                return carry

            lax.fori_loop(0, G, gbody, jnp.int32(0), unroll=2)

        def interp(level, colref, rowref):
            res = RES[level]
            s = np.float32(res - 1)

            def gbody(g, carry):
                o = g * jnp.int32(16)

                def weight1d(ref):
                    sc = ref[pl.ds(o, 16)] * s
                    return sc - sc.astype(jnp.int32).astype(jnp.float32)

                wx = weight1d(px)
                wy = weight1d(py)
                wz = weight1d(pz)
                one_f = np.float32(1.0)
                omx = one_f - wx
                omy = one_f - wy
                omz = one_f - wz
                one_i = jnp.int32(1)
                cf = []
                for c in range(8):
                    off = o + jnp.int32(c * CHUNK)
                    ridx = ids + off
                    colv = colref[pl.ds(off, 16)]
                    ca = colv + colv
                    cb = ca + one_i
                    cf.append((plsc.load_gather(rowref, [ridx, ca]),
                               plsc.load_gather(rowref, [ridx, cb])))
                pidx = ids + o
                for f in range(FEAT):
                    c00 = cf[0][f] * omx + cf[4][f] * wx
                    c01 = cf[1][f] * omx + cf[5][f] * wx
                    c10 = cf[2][f] * omx + cf[6][f] * wx
                    c11 = cf[3][f] * omx + cf[7][f] * wx
                    c0 = c00 * omy + c10 * wy
                    c1 = c01 * omy + c11 * wy
                    outv = c0 * omz + c1 * wz
                    col = jnp.full((16,), 2 * level + f, jnp.int32)
                    plsc.store_scatter(outb, [pidx, col], outv)
                return carry

            lax.fori_loop(0, G, gbody, jnp.int32(0), unroll=2)

        def chunk_body(ci, carry):
            base = wid * jnp.int32(PW) + ci * jnp.int32(CHUNK)
            pltpu.sync_copy(xs_h.at[pl.ds(base, CHUNK)], px)
            pltpu.sync_copy(ys_h.at[pl.ds(base, CHUNK)], py)
            pltpu.sync_copy(zs_h.at[pl.ds(base, CHUNK)], pz)

            def prep(g, c2):
                o = g * jnp.int32(16)
                for ref in (px, py, pz):
                    ref[pl.ds(o, 16)] = (
                        (ref[pl.ds(o, 16)] + np.float32(1.0)) * np.float32(0.5))
                return c2

            lax.fori_loop(jnp.int32(0), jnp.int32(G), prep, jnp.int32(0))

            compute_idx(0, idx0, col0b)
            cps = [None] * NUM_LEVELS
            cps[0] = pltpu.make_async_copy(tab_h.at[idx0], rows0, sem0)
            cps[0].start()
            for l in range(NUM_LEVELS):
                if l + 1 < NUM_LEVELS:
                    nb = (l + 1) % 2
                    compute_idx(l + 1, idxrefs[nb], colrefs[nb])
                    cps[l + 1] = pltpu.make_async_copy(
                        tab_h.at[idxrefs[nb]], rowrefs[nb], sems[nb])
                    cps[l + 1].start()
                b = l % 2
                cps[l].wait()
                interp(l, colrefs[b], rowrefs[b])
            pltpu.sync_copy(outb, out_h.at[pl.ds(base, CHUNK)])
            return carry

        lax.fori_loop(jnp.int32(0), jnp.int32(NCH), chunk_body, jnp.int32(0))

    return hash_enc


_KERNEL = _make_kernel()


def kernel(positions, hash_table):
    xs = positions[:, 0]
    ys = positions[:, 1]
    zs = positions[:, 2]
    tab = hash_table.reshape(NGRAN, GRAN)
    with jax.enable_x64(False):
        return _KERNEL(xs, ys, zs, tab)
